# baseline (speedup 1.0000x reference)
import functools

import jax
import jax.numpy as jnp
from jax import lax
from jax.experimental import pallas as pl
from jax.experimental.pallas import tpu as pltpu
from jax.experimental.pallas import tpu_sc as plsc

HID = 128
CH = 80
NC = 2
NS = 16
_BN_EPS = 1e-5
_INV = (1.0 + _BN_EPS) ** -0.5


def _make_sc_aggr(N, E):
    NW = NC * NS
    EPW = -(-E // NW // (16 * CH)) * 16 * CH
    NCH = EPW // CH
    NCHH = NCH // 2
    W = (N // NS // 8) * 8
    TAIL = N - W * NS
    ZB = 96
    mesh = plsc.VectorSubcoreMesh(core_axis_name="c", subcore_axis_name="s")

    @functools.partial(
        pl.kernel,
        mesh=mesh,
        out_type=jax.ShapeDtypeStruct((NC, N, HID), jnp.float32),
        scratch_types=[
            pltpu.VMEM((NCHH, CH), jnp.int32),
            pltpu.VMEM((NCHH, CH), jnp.int32),
            pltpu.VMEM((2, CH, HID), jnp.float32),
            pltpu.VMEM_SHARED((N + NW, HID), jnp.float32),
            pltpu.SemaphoreType.DMA,
            pltpu.SemaphoreType.DMA,
        ],
    )
    def aggr_k(h_hbm, src_hbm, dst_hbm, out_hbm,
               src_v, dst_v, rows_v, accum, sem0, sem1):
        cid = lax.axis_index("c")
        sid = lax.axis_index("s")
        wid = cid * NS + sid

        def zrow(i, carry):
            for k in range(HID // 16):
                rows_v[0, i, pl.ds(k * 16, 16)] = jnp.zeros((16,), jnp.float32)
            return carry
        lax.fori_loop(0, ZB, zrow, 0)
        for t in range(W // ZB):
            pltpu.sync_copy(rows_v.at[0, pl.ds(0, ZB)],
                            accum.at[pl.ds(sid * W + t * ZB, ZB)])
        ZT = W - (W // ZB) * ZB
        if ZT:
            pltpu.sync_copy(rows_v.at[0, pl.ds(0, ZT)],
                            accum.at[pl.ds(sid * W + (W // ZB) * ZB, ZT)])
        if TAIL:
            @pl.when(sid == 0)
            def _():
                pltpu.sync_copy(rows_v.at[0, pl.ds(0, TAIL)],
                                accum.at[pl.ds(N - TAIL, TAIL)])
        plsc.subcore_barrier()

        for half in range(2):
            base = half * NCHH
            pltpu.sync_copy(src_hbm.at[wid, pl.ds(base, NCHH)], src_v)
            pltpu.sync_copy(dst_hbm.at[wid, pl.ds(base, NCHH)], dst_v)
            def chunk(j, carry):
                pltpu.async_copy(h_hbm.at[src_v.at[j]],
                                 rows_v.at[0], sem0).wait()
                pltpu.sync_copy(rows_v.at[0], accum.at[dst_v.at[j]], add=True)
                return carry
            lax.fori_loop(0, NCHH, chunk, 0)

        plsc.subcore_barrier()
        pltpu.sync_copy(accum.at[pl.ds(sid * W, W)],
                        out_hbm.at[cid, pl.ds(sid * W, W)])
        if TAIL:
            @pl.when(sid == 0)
            def _():
                pltpu.sync_copy(accum.at[pl.ds(N - TAIL, TAIL)],
                                out_hbm.at[cid, pl.ds(N - TAIL, TAIL)])

    return aggr_k


def _dot_t(x, w):
    return lax.dot_general(x, w, (((1,), (1,)), ((), ())),
                           preferred_element_type=jnp.float32)


def _tc0_body(s_ref, emb_ref, h0_ref):
    s = s_ref[...][0]
    rb = s.shape[0]
    emb = emb_ref[...]
    h0 = (1.0 - s) * emb[0:1, :] + s * emb[1:2, :]
    h0_ref[...] = h0.reshape(1, rb, HID)


def _tc0(s3, emb):
    nb, rb = s3.shape[0], s3.shape[1]
    return pl.pallas_call(
        _tc0_body,
        grid=(nb,),
        in_specs=[
            pl.BlockSpec((1, rb, 1), lambda i: (i, 0, 0)),
            pl.BlockSpec((2, HID), lambda i: (0, 0)),
        ],
        out_specs=pl.BlockSpec((1, rb, HID), lambda i: (i, 0, 0)),
        out_shape=jax.ShapeDtypeStruct((nb, rb, HID), jnp.float32),
    )(s3, emb)


def _gin_mlp(z, w1, g1, b1, w2, g2, b2):
    t = _dot_t(z, w1)
    t = jnp.maximum(t * _INV * g1 + b1, 0.0)
    z2 = _dot_t(t, w2)
    return jnp.maximum(z2 * _INV * g2 + b2, 0.0)


def _tc1_body(h_ref, parts_ref, w1_ref, g1_ref, b1_ref,
              w2_ref, g2_ref, b2_ref, out_ref):
    h = h_ref[...][0]
    rb = h.shape[0]
    p = parts_ref[...]
    z = h + p[0, 0] + p[1, 0]
    out = _gin_mlp(z, w1_ref[...], g1_ref[...], b1_ref[...],
                   w2_ref[...], g2_ref[...], b2_ref[...])
    out_ref[...] = out.reshape(1, rb, HID)


def _tc1(h, parts, w1, g1, b1, w2, g2, b2):
    nb, rb = h.shape[0], h.shape[1]
    full = lambda shape: pl.BlockSpec(shape, lambda i: tuple(0 for _ in shape))
    return pl.pallas_call(
        _tc1_body,
        grid=(nb,),
        in_specs=[
            pl.BlockSpec((1, rb, HID), lambda i: (i, 0, 0)),
            pl.BlockSpec((NC, 1, rb, HID), lambda i: (0, i, 0, 0)),
            full((HID, HID)), full((1, HID)), full((1, HID)),
            full((HID, HID)), full((1, HID)), full((1, HID)),
        ],
        out_specs=pl.BlockSpec((1, rb, HID), lambda i: (i, 0, 0)),
        out_shape=jax.ShapeDtypeStruct((nb, rb, HID), jnp.float32),
    )(h, parts, w1, g1, b1, w2, g2, b2)


def _tc2_body(h0_ref, h1_ref, parts_ref, w1_ref, g1_ref, b1_ref,
              w2_ref, g2_ref, b2_ref, r0_ref, r1_ref, r2_ref,
              rb1_ref, rw2_ref, rb2_ref, sc_ref, pool_ref):
    h0 = h0_ref[...][0]
    h1 = h1_ref[...][0]
    rb = h1.shape[0]
    p = parts_ref[...]
    z = h1 + p[0, 0] + p[1, 0]
    h2 = _gin_mlp(z, w1_ref[...], g1_ref[...], b1_ref[...],
                  w2_ref[...], g2_ref[...], b2_ref[...])
    pre = (_dot_t(h0, r0_ref[...]) + _dot_t(h1, r1_ref[...])
           + _dot_t(h2, r2_ref[...]) + rb1_ref[...])
    pre = jnp.maximum(pre, 0.0)
    q = _dot_t(pre, rw2_ref[...])[:, 0:1] + rb2_ref[0, 0]
    sc_ref[...] = q.reshape(1, rb, 1)
    pool_ref[...] = jnp.max(q).reshape(1, 1, 1)


def _tc2(h0, h1, parts, w1, g1, b1, w2, g2, b2, r0, r1, r2, rb1, rw2, rb2):
    nb, rb = h1.shape[0], h1.shape[1]
    full = lambda shape: pl.BlockSpec(shape, lambda i: tuple(0 for _ in shape))
    return pl.pallas_call(
        _tc2_body,
        grid=(nb,),
        in_specs=[
            pl.BlockSpec((1, rb, HID), lambda i: (i, 0, 0)),
            pl.BlockSpec((1, rb, HID), lambda i: (i, 0, 0)),
            pl.BlockSpec((NC, 1, rb, HID), lambda i: (0, i, 0, 0)),
            full((HID, HID)), full((1, HID)), full((1, HID)),
            full((HID, HID)), full((1, HID)), full((1, HID)),
            full((HID, HID)), full((HID, HID)), full((HID, HID)),
            full((1, HID)), full((8, HID)), full((1, 1)),
        ],
        out_specs=[
            pl.BlockSpec((1, rb, 1), lambda i: (i, 0, 0)),
            pl.BlockSpec((1, 1, 1), lambda i: (i, 0, 0)),
        ],
        out_shape=[
            jax.ShapeDtypeStruct((nb, rb, 1), jnp.float32),
            jax.ShapeDtypeStruct((nb, 1, 1), jnp.float32),
        ],
    )(h0, h1, parts, w1, g1, b1, w2, g2, b2, r0, r1, r2, rb1, rw2, rb2)


def kernel(state, edge_index, params):
    b, npg = state.shape
    n = b * npg
    e = edge_index.shape[1]
    s3 = state.reshape(b, npg, 1).astype(jnp.float32)
    nw = NC * NS
    epw = e // nw
    epw_pad = -(-epw // (16 * CH)) * 16 * CH
    src2 = jnp.pad(edge_index[0].reshape(nw, epw),
                   ((0, 0), (0, epw_pad - epw))).reshape(nw, -1, CH)
    dump = jnp.broadcast_to(n + jnp.arange(nw, dtype=jnp.int32)[:, None],
                            (nw, epw_pad - epw))
    dst2 = jnp.concatenate([edge_index[1].reshape(nw, epw), dump],
                           axis=1).reshape(nw, -1, CH)

    l0, l1 = params["layers"][0], params["layers"][1]
    emb = params["emb"]
    row = lambda v: v.reshape(1, -1)
    sweep = _make_sc_aggr(n, e)

    h0 = _tc0(s3, emb)
    p0 = sweep(h0.reshape(n, HID), src2, dst2)
    h1 = _tc1(h0, p0.reshape(NC, b, npg, HID),
              l0["W1"], row(l0["bn1_g"]), row(l0["bn1_b"]),
              l0["W2"], row(l0["obn_g"]), row(l0["obn_b"]))
    p1 = sweep(h1.reshape(n, HID), src2, dst2)
    rw1 = params["r_W1"]
    scores3, pooled3 = _tc2(
        h0, h1, p1.reshape(NC, b, npg, HID),
        l1["W1"], row(l1["bn1_g"]), row(l1["bn1_b"]),
        l1["W2"], row(l1["obn_g"]), row(l1["obn_b"]),
        rw1[:, :HID], rw1[:, HID:2 * HID], rw1[:, 2 * HID:],
        row(params["r_b1"]), jnp.pad(params["r_W2"], ((0, 7), (0, 0))),
        params["r_b2"].reshape(1, 1))
    return scores3.reshape(b, npg), pooled3.reshape(b, 1)


# speedup vs baseline: 2.2678x; 2.2678x over previous
import functools

import jax
import jax.numpy as jnp
from jax import lax
from jax.experimental import pallas as pl
from jax.experimental.pallas import tpu as pltpu
from jax.experimental.pallas import tpu_sc as plsc

HID = 128
CH = 100
NC = 2
NS = 16
_BN_EPS = 1e-5
_INV = (1.0 + _BN_EPS) ** -0.5


def _make_sc_aggr(N, E):
    NW = NC * NS
    EPW = E // NW
    NCH = EPW // CH
    W = (N // NS // 8) * 8
    TAIL = N - W * NS
    ZR = 48
    mesh = plsc.VectorSubcoreMesh(core_axis_name="c", subcore_axis_name="s")

    @functools.partial(
        pl.kernel,
        mesh=mesh,
        out_type=jax.ShapeDtypeStruct((NC, N, HID), jnp.float32),
        scratch_types=[
            pltpu.VMEM((NCH, CH), jnp.int32),
            pltpu.VMEM((NCH, CH), jnp.int32),
            pltpu.VMEM((CH, HID), jnp.float32),
            pltpu.VMEM((ZR, HID), jnp.float32),
            pltpu.VMEM_SHARED((N, HID), jnp.float32),
            pltpu.SemaphoreType.DMA,
        ],
    )
    def aggr_k(h_hbm, src_hbm, dst_hbm, out_hbm,
               src_v, dst_v, rows_v, zero_v, accum, sem):
        cid = lax.axis_index("c")
        sid = lax.axis_index("s")
        wid = cid * NS + sid

        def zrow(i, carry):
            for k in range(HID // 16):
                zero_v[i, pl.ds(k * 16, 16)] = jnp.zeros((16,), jnp.float32)
            return carry
        lax.fori_loop(0, ZR, zrow, 0)
        for t in range(W // ZR):
            pltpu.sync_copy(zero_v, accum.at[pl.ds(sid * W + t * ZR, ZR)])
        if TAIL:
            @pl.when(sid == 0)
            def _():
                pltpu.sync_copy(zero_v.at[pl.ds(0, TAIL)],
                                accum.at[pl.ds(N - TAIL, TAIL)])

        pltpu.sync_copy(src_hbm.at[wid], src_v)
        pltpu.sync_copy(dst_hbm.at[wid], dst_v)
        plsc.subcore_barrier()

        def chunk(j, carry):
            pltpu.async_copy(h_hbm.at[src_v.at[j]], rows_v, sem).wait()
            pltpu.sync_copy(rows_v, accum.at[dst_v.at[j]], add=True)
            return carry
        lax.fori_loop(0, NCH, chunk, 0)

        plsc.subcore_barrier()
        pltpu.sync_copy(accum.at[pl.ds(sid * W, W)],
                        out_hbm.at[cid, pl.ds(sid * W, W)])
        if TAIL:
            @pl.when(sid == 0)
            def _():
                pltpu.sync_copy(accum.at[pl.ds(N - TAIL, TAIL)],
                                out_hbm.at[cid, pl.ds(N - TAIL, TAIL)])

    return aggr_k


def _dot_t(x, w):
    return lax.dot_general(x, w, (((1,), (1,)), ((), ())),
                           preferred_element_type=jnp.float32)


def _tc0_body(s_ref, emb_ref, h0_ref):
    s = s_ref[...][0]
    rb = s.shape[0]
    emb = emb_ref[...]
    h0 = (1.0 - s) * emb[0:1, :] + s * emb[1:2, :]
    h0_ref[...] = h0.reshape(1, rb, HID)


def _tc0(s3, emb):
    nb, rb = s3.shape[0], s3.shape[1]
    return pl.pallas_call(
        _tc0_body,
        grid=(nb,),
        in_specs=[
            pl.BlockSpec((1, rb, 1), lambda i: (i, 0, 0)),
            pl.BlockSpec((2, HID), lambda i: (0, 0)),
        ],
        out_specs=pl.BlockSpec((1, rb, HID), lambda i: (i, 0, 0)),
        out_shape=jax.ShapeDtypeStruct((nb, rb, HID), jnp.float32),
    )(s3, emb)


def _gin_mlp(z, w1, g1, b1, w2, g2, b2):
    t = _dot_t(z, w1)
    t = jnp.maximum(t * _INV * g1 + b1, 0.0)
    z2 = _dot_t(t, w2)
    return jnp.maximum(z2 * _INV * g2 + b2, 0.0)


def _tc1_body(h_ref, parts_ref, w1_ref, g1_ref, b1_ref,
              w2_ref, g2_ref, b2_ref, out_ref):
    h = h_ref[...][0]
    rb = h.shape[0]
    p = parts_ref[...]
    z = h + p[0, 0] + p[1, 0]
    out = _gin_mlp(z, w1_ref[...], g1_ref[...], b1_ref[...],
                   w2_ref[...], g2_ref[...], b2_ref[...])
    out_ref[...] = out.reshape(1, rb, HID)


def _tc1(h, parts, w1, g1, b1, w2, g2, b2):
    nb, rb = h.shape[0], h.shape[1]
    full = lambda shape: pl.BlockSpec(shape, lambda i: tuple(0 for _ in shape))
    return pl.pallas_call(
        _tc1_body,
        grid=(nb,),
        in_specs=[
            pl.BlockSpec((1, rb, HID), lambda i: (i, 0, 0)),
            pl.BlockSpec((NC, 1, rb, HID), lambda i: (0, i, 0, 0)),
            full((HID, HID)), full((1, HID)), full((1, HID)),
            full((HID, HID)), full((1, HID)), full((1, HID)),
        ],
        out_specs=pl.BlockSpec((1, rb, HID), lambda i: (i, 0, 0)),
        out_shape=jax.ShapeDtypeStruct((nb, rb, HID), jnp.float32),
    )(h, parts, w1, g1, b1, w2, g2, b2)


def _tc2_body(h0_ref, h1_ref, parts_ref, w1_ref, g1_ref, b1_ref,
              w2_ref, g2_ref, b2_ref, r0_ref, r1_ref, r2_ref,
              rb1_ref, rw2_ref, rb2_ref, sc_ref, pool_ref):
    h0 = h0_ref[...][0]
    h1 = h1_ref[...][0]
    rb = h1.shape[0]
    p = parts_ref[...]
    z = h1 + p[0, 0] + p[1, 0]
    h2 = _gin_mlp(z, w1_ref[...], g1_ref[...], b1_ref[...],
                  w2_ref[...], g2_ref[...], b2_ref[...])
    pre = (_dot_t(h0, r0_ref[...]) + _dot_t(h1, r1_ref[...])
           + _dot_t(h2, r2_ref[...]) + rb1_ref[...])
    pre = jnp.maximum(pre, 0.0)
    q = _dot_t(pre, rw2_ref[...])[:, 0:1] + rb2_ref[0, 0]
    sc_ref[...] = q.reshape(1, rb, 1)
    pool_ref[...] = jnp.max(q).reshape(1, 1, 1)


def _tc2(h0, h1, parts, w1, g1, b1, w2, g2, b2, r0, r1, r2, rb1, rw2, rb2):
    nb, rb = h1.shape[0], h1.shape[1]
    full = lambda shape: pl.BlockSpec(shape, lambda i: tuple(0 for _ in shape))
    return pl.pallas_call(
        _tc2_body,
        grid=(nb,),
        in_specs=[
            pl.BlockSpec((1, rb, HID), lambda i: (i, 0, 0)),
            pl.BlockSpec((1, rb, HID), lambda i: (i, 0, 0)),
            pl.BlockSpec((NC, 1, rb, HID), lambda i: (0, i, 0, 0)),
            full((HID, HID)), full((1, HID)), full((1, HID)),
            full((HID, HID)), full((1, HID)), full((1, HID)),
            full((HID, HID)), full((HID, HID)), full((HID, HID)),
            full((1, HID)), full((8, HID)), full((1, 1)),
        ],
        out_specs=[
            pl.BlockSpec((1, rb, 1), lambda i: (i, 0, 0)),
            pl.BlockSpec((1, 1, 1), lambda i: (i, 0, 0)),
        ],
        out_shape=[
            jax.ShapeDtypeStruct((nb, rb, 1), jnp.float32),
            jax.ShapeDtypeStruct((nb, 1, 1), jnp.float32),
        ],
    )(h0, h1, parts, w1, g1, b1, w2, g2, b2, r0, r1, r2, rb1, rw2, rb2)


def kernel(state, edge_index, params):
    b, npg = state.shape
    n = b * npg
    e = edge_index.shape[1]
    s3 = state.reshape(b, npg, 1).astype(jnp.float32)
    nw = NC * NS
    src2 = edge_index[0].reshape(nw, e // (nw * CH), CH)
    dst2 = edge_index[1].reshape(nw, e // (nw * CH), CH)

    l0, l1 = params["layers"][0], params["layers"][1]
    emb = params["emb"]
    row = lambda v: v.reshape(1, -1)
    sweep = _make_sc_aggr(n, e)

    h0 = _tc0(s3, emb)
    p0 = sweep(h0.reshape(n, HID), src2, dst2)
    h1 = _tc1(h0, p0.reshape(NC, b, npg, HID),
              l0["W1"], row(l0["bn1_g"]), row(l0["bn1_b"]),
              l0["W2"], row(l0["obn_g"]), row(l0["obn_b"]))
    p1 = sweep(h1.reshape(n, HID), src2, dst2)
    rw1 = params["r_W1"]
    scores3, pooled3 = _tc2(
        h0, h1, p1.reshape(NC, b, npg, HID),
        l1["W1"], row(l1["bn1_g"]), row(l1["bn1_b"]),
        l1["W2"], row(l1["obn_g"]), row(l1["obn_b"]),
        rw1[:, :HID], rw1[:, HID:2 * HID], rw1[:, 2 * HID:],
        row(params["r_b1"]), jnp.pad(params["r_W2"], ((0, 7), (0, 0))),
        params["r_b2"].reshape(1, 1))
    return scores3.reshape(b, npg), pooled3.reshape(b, 1)


# speedup vs baseline: 2.4052x; 1.0606x over previous
import functools

import jax
import jax.numpy as jnp
from jax import lax
from jax.experimental import pallas as pl
from jax.experimental.pallas import tpu as pltpu
from jax.experimental.pallas import tpu_sc as plsc

HID = 128
CH = 125
NC = 2
NS = 16
_BN_EPS = 1e-5
_INV = (1.0 + _BN_EPS) ** -0.5


def _make_sc_aggr(N, E):
    NW = NC * NS
    EPW = E // NW
    NCH = EPW // CH
    W = (N // NS // 8) * 8
    TAIL = N - W * NS
    ZR = 48
    mesh = plsc.VectorSubcoreMesh(core_axis_name="c", subcore_axis_name="s")

    @functools.partial(
        pl.kernel,
        mesh=mesh,
        out_type=jax.ShapeDtypeStruct((NC, N, HID), jnp.float32),
        scratch_types=[
            pltpu.VMEM((NCH, CH), jnp.int32),
            pltpu.VMEM((NCH, CH), jnp.int32),
            pltpu.VMEM((CH, HID), jnp.float32),
            pltpu.VMEM((ZR, HID), jnp.float32),
            pltpu.VMEM_SHARED((N, HID), jnp.float32),
            pltpu.SemaphoreType.DMA,
        ],
    )
    def aggr_k(h_hbm, src_hbm, dst_hbm, out_hbm,
               src_v, dst_v, rows_v, zero_v, accum, sem):
        cid = lax.axis_index("c")
        sid = lax.axis_index("s")
        wid = cid * NS + sid

        def zrow(i, carry):
            for k in range(HID // 16):
                zero_v[i, pl.ds(k * 16, 16)] = jnp.zeros((16,), jnp.float32)
            return carry
        lax.fori_loop(0, ZR, zrow, 0)
        for t in range(W // ZR):
            pltpu.sync_copy(zero_v, accum.at[pl.ds(sid * W + t * ZR, ZR)])
        if TAIL:
            @pl.when(sid == 0)
            def _():
                pltpu.sync_copy(zero_v.at[pl.ds(0, TAIL)],
                                accum.at[pl.ds(N - TAIL, TAIL)])

        pltpu.sync_copy(src_hbm.at[wid], src_v)
        pltpu.sync_copy(dst_hbm.at[wid], dst_v)
        plsc.subcore_barrier()

        def chunk(j, carry):
            pltpu.async_copy(h_hbm.at[src_v.at[j]], rows_v, sem).wait()
            pltpu.sync_copy(rows_v, accum.at[dst_v.at[j]], add=True)
            return carry
        lax.fori_loop(0, NCH, chunk, 0)

        plsc.subcore_barrier()
        pltpu.sync_copy(accum.at[pl.ds(sid * W, W)],
                        out_hbm.at[cid, pl.ds(sid * W, W)])
        if TAIL:
            @pl.when(sid == 0)
            def _():
                pltpu.sync_copy(accum.at[pl.ds(N - TAIL, TAIL)],
                                out_hbm.at[cid, pl.ds(N - TAIL, TAIL)])

    return aggr_k


def _dot_t(x, w):
    return lax.dot_general(x, w, (((1,), (1,)), ((), ())),
                           preferred_element_type=jnp.float32)


def _tc0_body(s_ref, emb_ref, h0_ref):
    s = s_ref[...][0]
    rb = s.shape[0]
    emb = emb_ref[...]
    h0 = (1.0 - s) * emb[0:1, :] + s * emb[1:2, :]
    h0_ref[...] = h0.reshape(1, rb, HID)


def _tc0(s3, emb):
    nb, rb = s3.shape[0], s3.shape[1]
    return pl.pallas_call(
        _tc0_body,
        grid=(nb,),
        in_specs=[
            pl.BlockSpec((1, rb, 1), lambda i: (i, 0, 0)),
            pl.BlockSpec((2, HID), lambda i: (0, 0)),
        ],
        out_specs=pl.BlockSpec((1, rb, HID), lambda i: (i, 0, 0)),
        out_shape=jax.ShapeDtypeStruct((nb, rb, HID), jnp.float32),
    )(s3, emb)


def _gin_mlp(z, w1, g1, b1, w2, g2, b2):
    t = _dot_t(z, w1)
    t = jnp.maximum(t * _INV * g1 + b1, 0.0)
    z2 = _dot_t(t, w2)
    return jnp.maximum(z2 * _INV * g2 + b2, 0.0)


def _tc1_body(h_ref, parts_ref, w1_ref, g1_ref, b1_ref,
              w2_ref, g2_ref, b2_ref, out_ref):
    h = h_ref[...][0]
    rb = h.shape[0]
    p = parts_ref[...]
    z = h + p[0, 0] + p[1, 0]
    out = _gin_mlp(z, w1_ref[...], g1_ref[...], b1_ref[...],
                   w2_ref[...], g2_ref[...], b2_ref[...])
    out_ref[...] = out.reshape(1, rb, HID)


def _tc1(h, parts, w1, g1, b1, w2, g2, b2):
    nb, rb = h.shape[0], h.shape[1]
    full = lambda shape: pl.BlockSpec(shape, lambda i: tuple(0 for _ in shape))
    return pl.pallas_call(
        _tc1_body,
        grid=(nb,),
        in_specs=[
            pl.BlockSpec((1, rb, HID), lambda i: (i, 0, 0)),
            pl.BlockSpec((NC, 1, rb, HID), lambda i: (0, i, 0, 0)),
            full((HID, HID)), full((1, HID)), full((1, HID)),
            full((HID, HID)), full((1, HID)), full((1, HID)),
        ],
        out_specs=pl.BlockSpec((1, rb, HID), lambda i: (i, 0, 0)),
        out_shape=jax.ShapeDtypeStruct((nb, rb, HID), jnp.float32),
    )(h, parts, w1, g1, b1, w2, g2, b2)


def _tc2_body(h0_ref, h1_ref, parts_ref, w1_ref, g1_ref, b1_ref,
              w2_ref, g2_ref, b2_ref, r0_ref, r1_ref, r2_ref,
              rb1_ref, rw2_ref, rb2_ref, sc_ref, pool_ref):
    h0 = h0_ref[...][0]
    h1 = h1_ref[...][0]
    rb = h1.shape[0]
    p = parts_ref[...]
    z = h1 + p[0, 0] + p[1, 0]
    h2 = _gin_mlp(z, w1_ref[...], g1_ref[...], b1_ref[...],
                  w2_ref[...], g2_ref[...], b2_ref[...])
    pre = (_dot_t(h0, r0_ref[...]) + _dot_t(h1, r1_ref[...])
           + _dot_t(h2, r2_ref[...]) + rb1_ref[...])
    pre = jnp.maximum(pre, 0.0)
    q = _dot_t(pre, rw2_ref[...])[:, 0:1] + rb2_ref[0, 0]
    sc_ref[...] = q.reshape(1, rb, 1)
    pool_ref[...] = jnp.max(q).reshape(1, 1, 1)


def _tc2(h0, h1, parts, w1, g1, b1, w2, g2, b2, r0, r1, r2, rb1, rw2, rb2):
    nb, rb = h1.shape[0], h1.shape[1]
    full = lambda shape: pl.BlockSpec(shape, lambda i: tuple(0 for _ in shape))
    return pl.pallas_call(
        _tc2_body,
        grid=(nb,),
        in_specs=[
            pl.BlockSpec((1, rb, HID), lambda i: (i, 0, 0)),
            pl.BlockSpec((1, rb, HID), lambda i: (i, 0, 0)),
            pl.BlockSpec((NC, 1, rb, HID), lambda i: (0, i, 0, 0)),
            full((HID, HID)), full((1, HID)), full((1, HID)),
            full((HID, HID)), full((1, HID)), full((1, HID)),
            full((HID, HID)), full((HID, HID)), full((HID, HID)),
            full((1, HID)), full((8, HID)), full((1, 1)),
        ],
        out_specs=[
            pl.BlockSpec((1, rb, 1), lambda i: (i, 0, 0)),
            pl.BlockSpec((1, 1, 1), lambda i: (i, 0, 0)),
        ],
        out_shape=[
            jax.ShapeDtypeStruct((nb, rb, 1), jnp.float32),
            jax.ShapeDtypeStruct((nb, 1, 1), jnp.float32),
        ],
    )(h0, h1, parts, w1, g1, b1, w2, g2, b2, r0, r1, r2, rb1, rw2, rb2)


def kernel(state, edge_index, params):
    b, npg = state.shape
    n = b * npg
    e = edge_index.shape[1]
    s3 = state.reshape(b, npg, 1).astype(jnp.float32)
    nw = NC * NS
    src2 = edge_index[0].reshape(nw, e // (nw * CH), CH)
    dst2 = edge_index[1].reshape(nw, e // (nw * CH), CH)

    l0, l1 = params["layers"][0], params["layers"][1]
    emb = params["emb"]
    row = lambda v: v.reshape(1, -1)
    sweep = _make_sc_aggr(n, e)

    h0 = _tc0(s3, emb)
    p0 = sweep(h0.reshape(n, HID), src2, dst2)
    h1 = _tc1(h0, p0.reshape(NC, b, npg, HID),
              l0["W1"], row(l0["bn1_g"]), row(l0["bn1_b"]),
              l0["W2"], row(l0["obn_g"]), row(l0["obn_b"]))
    p1 = sweep(h1.reshape(n, HID), src2, dst2)
    rw1 = params["r_W1"]
    scores3, pooled3 = _tc2(
        h0, h1, p1.reshape(NC, b, npg, HID),
        l1["W1"], row(l1["bn1_g"]), row(l1["bn1_b"]),
        l1["W2"], row(l1["obn_g"]), row(l1["obn_b"]),
        rw1[:, :HID], rw1[:, HID:2 * HID], rw1[:, 2 * HID:],
        row(params["r_b1"]), jnp.pad(params["r_W2"], ((0, 7), (0, 0))),
        params["r_b2"].reshape(1, 1))
    return scores3.reshape(b, npg), pooled3.reshape(b, 1)


# speedup vs baseline: 3.4166x; 1.4205x over previous
import functools

import jax
import jax.numpy as jnp
from jax import lax
from jax.experimental import pallas as pl
from jax.experimental.pallas import tpu as pltpu
from jax.experimental.pallas import tpu_sc as plsc

HID = 128
CH = 125
NC = 2
NS = 16
_BN_EPS = 1e-5
_INV = (1.0 + _BN_EPS) ** -0.5


def _make_sc_aggr(N, E):
    NW = NC * NS
    EPW = E // NW
    NCH = EPW // CH
    NCHH = NCH // 2
    W = (N // NS // 8) * 8
    TAIL = N - W * NS
    ZR = 48
    mesh = plsc.VectorSubcoreMesh(core_axis_name="c", subcore_axis_name="s")

    @functools.partial(
        pl.kernel,
        mesh=mesh,
        out_type=jax.ShapeDtypeStruct((NC, N, HID), jnp.float32),
        scratch_types=[
            pltpu.VMEM((NCHH, CH), jnp.int32),
            pltpu.VMEM((NCHH, CH), jnp.int32),
            pltpu.VMEM((CH, HID), jnp.float32),
            pltpu.VMEM((CH, HID), jnp.float32),
            pltpu.VMEM((ZR, HID), jnp.float32),
            pltpu.VMEM_SHARED((N, HID), jnp.float32),
            pltpu.SemaphoreType.DMA,
            pltpu.SemaphoreType.DMA,
        ],
    )
    def aggr_k(h_hbm, src_hbm, dst_hbm, out_hbm,
               src_v, dst_v, rows_a, rows_b, zero_v, accum, sem_a, sem_b):
        cid = lax.axis_index("c")
        sid = lax.axis_index("s")
        wid = cid * NS + sid

        def zrow(i, carry):
            for k in range(HID // 16):
                zero_v[i, pl.ds(k * 16, 16)] = jnp.zeros((16,), jnp.float32)
            return carry
        lax.fori_loop(0, ZR, zrow, 0)
        for t in range(W // ZR):
            pltpu.sync_copy(zero_v, accum.at[pl.ds(sid * W + t * ZR, ZR)])
        if TAIL:
            @pl.when(sid == 0)
            def _():
                pltpu.sync_copy(zero_v.at[pl.ds(0, TAIL)],
                                accum.at[pl.ds(N - TAIL, TAIL)])

        plsc.subcore_barrier()

        assert NCHH % 2 == 0 and NCHH >= 4
        for half in range(2):
            pltpu.sync_copy(src_hbm.at[wid, pl.ds(half * NCHH, NCHH)], src_v)
            pltpu.sync_copy(dst_hbm.at[wid, pl.ds(half * NCHH, NCHH)], dst_v)
            pltpu.async_copy(h_hbm.at[src_v.at[0]], rows_a, sem_a)

            def pair(i, carry):
                j = 2 * i
                pltpu.async_copy(h_hbm.at[src_v.at[j + 1]], rows_b, sem_b)
                pltpu.make_async_copy(h_hbm.at[src_v.at[j]],
                                      rows_a, sem_a).wait()
                pltpu.sync_copy(rows_a, accum.at[dst_v.at[j]], add=True)

                @pl.when(j + 2 < NCHH)
                def _():
                    pltpu.async_copy(h_hbm.at[src_v.at[j + 2]], rows_a, sem_a)
                pltpu.make_async_copy(h_hbm.at[src_v.at[j + 1]],
                                      rows_b, sem_b).wait()
                pltpu.sync_copy(rows_b, accum.at[dst_v.at[j + 1]], add=True)
                return carry
            lax.fori_loop(0, NCHH // 2, pair, 0)

        plsc.subcore_barrier()
        pltpu.sync_copy(accum.at[pl.ds(sid * W, W)],
                        out_hbm.at[cid, pl.ds(sid * W, W)])
        if TAIL:
            @pl.when(sid == 0)
            def _():
                pltpu.sync_copy(accum.at[pl.ds(N - TAIL, TAIL)],
                                out_hbm.at[cid, pl.ds(N - TAIL, TAIL)])

    return aggr_k


def _dot_t(x, w):
    return lax.dot_general(x, w, (((1,), (1,)), ((), ())),
                           preferred_element_type=jnp.float32)


def _tc0_body(s_ref, emb_ref, h0_ref):
    s = s_ref[...][0]
    rb = s.shape[0]
    emb = emb_ref[...]
    h0 = (1.0 - s) * emb[0:1, :] + s * emb[1:2, :]
    h0_ref[...] = h0.reshape(1, rb, HID)


def _tc0(s3, emb):
    nb, rb = s3.shape[0], s3.shape[1]
    return pl.pallas_call(
        _tc0_body,
        grid=(nb,),
        in_specs=[
            pl.BlockSpec((1, rb, 1), lambda i: (i, 0, 0)),
            pl.BlockSpec((2, HID), lambda i: (0, 0)),
        ],
        out_specs=pl.BlockSpec((1, rb, HID), lambda i: (i, 0, 0)),
        out_shape=jax.ShapeDtypeStruct((nb, rb, HID), jnp.float32),
    )(s3, emb)


def _gin_mlp(z, w1, g1, b1, w2, g2, b2):
    t = _dot_t(z, w1)
    t = jnp.maximum(t * _INV * g1 + b1, 0.0)
    z2 = _dot_t(t, w2)
    return jnp.maximum(z2 * _INV * g2 + b2, 0.0)


def _tc1_body(h_ref, parts_ref, w1_ref, g1_ref, b1_ref,
              w2_ref, g2_ref, b2_ref, out_ref):
    h = h_ref[...][0]
    rb = h.shape[0]
    p = parts_ref[...]
    z = h + p[0, 0] + p[1, 0]
    out = _gin_mlp(z, w1_ref[...], g1_ref[...], b1_ref[...],
                   w2_ref[...], g2_ref[...], b2_ref[...])
    out_ref[...] = out.reshape(1, rb, HID)


def _tc1(h, parts, w1, g1, b1, w2, g2, b2):
    nb, rb = h.shape[0], h.shape[1]
    full = lambda shape: pl.BlockSpec(shape, lambda i: tuple(0 for _ in shape))
    return pl.pallas_call(
        _tc1_body,
        grid=(nb,),
        in_specs=[
            pl.BlockSpec((1, rb, HID), lambda i: (i, 0, 0)),
            pl.BlockSpec((NC, 1, rb, HID), lambda i: (0, i, 0, 0)),
            full((HID, HID)), full((1, HID)), full((1, HID)),
            full((HID, HID)), full((1, HID)), full((1, HID)),
        ],
        out_specs=pl.BlockSpec((1, rb, HID), lambda i: (i, 0, 0)),
        out_shape=jax.ShapeDtypeStruct((nb, rb, HID), jnp.float32),
    )(h, parts, w1, g1, b1, w2, g2, b2)


def _tc2_body(h0_ref, h1_ref, parts_ref, w1_ref, g1_ref, b1_ref,
              w2_ref, g2_ref, b2_ref, r0_ref, r1_ref, r2_ref,
              rb1_ref, rw2_ref, rb2_ref, sc_ref, pool_ref):
    h0 = h0_ref[...][0]
    h1 = h1_ref[...][0]
    rb = h1.shape[0]
    p = parts_ref[...]
    z = h1 + p[0, 0] + p[1, 0]
    h2 = _gin_mlp(z, w1_ref[...], g1_ref[...], b1_ref[...],
                  w2_ref[...], g2_ref[...], b2_ref[...])
    pre = (_dot_t(h0, r0_ref[...]) + _dot_t(h1, r1_ref[...])
           + _dot_t(h2, r2_ref[...]) + rb1_ref[...])
    pre = jnp.maximum(pre, 0.0)
    q = _dot_t(pre, rw2_ref[...])[:, 0:1] + rb2_ref[0, 0]
    sc_ref[...] = q.reshape(1, rb, 1)
    pool_ref[...] = jnp.max(q).reshape(1, 1, 1)


def _tc2(h0, h1, parts, w1, g1, b1, w2, g2, b2, r0, r1, r2, rb1, rw2, rb2):
    nb, rb = h1.shape[0], h1.shape[1]
    full = lambda shape: pl.BlockSpec(shape, lambda i: tuple(0 for _ in shape))
    return pl.pallas_call(
        _tc2_body,
        grid=(nb,),
        in_specs=[
            pl.BlockSpec((1, rb, HID), lambda i: (i, 0, 0)),
            pl.BlockSpec((1, rb, HID), lambda i: (i, 0, 0)),
            pl.BlockSpec((NC, 1, rb, HID), lambda i: (0, i, 0, 0)),
            full((HID, HID)), full((1, HID)), full((1, HID)),
            full((HID, HID)), full((1, HID)), full((1, HID)),
            full((HID, HID)), full((HID, HID)), full((HID, HID)),
            full((1, HID)), full((8, HID)), full((1, 1)),
        ],
        out_specs=[
            pl.BlockSpec((1, rb, 1), lambda i: (i, 0, 0)),
            pl.BlockSpec((1, 1, 1), lambda i: (i, 0, 0)),
        ],
        out_shape=[
            jax.ShapeDtypeStruct((nb, rb, 1), jnp.float32),
            jax.ShapeDtypeStruct((nb, 1, 1), jnp.float32),
        ],
    )(h0, h1, parts, w1, g1, b1, w2, g2, b2, r0, r1, r2, rb1, rw2, rb2)


def kernel(state, edge_index, params):
    b, npg = state.shape
    n = b * npg
    e = edge_index.shape[1]
    s3 = state.reshape(b, npg, 1).astype(jnp.float32)
    nw = NC * NS
    src2 = edge_index[0].reshape(nw, e // (nw * CH), CH)
    dst2 = edge_index[1].reshape(nw, e // (nw * CH), CH)

    l0, l1 = params["layers"][0], params["layers"][1]
    emb = params["emb"]
    row = lambda v: v.reshape(1, -1)
    sweep = _make_sc_aggr(n, e)

    h0 = _tc0(s3, emb)
    p0 = sweep(h0.reshape(n, HID), src2, dst2)
    h1 = _tc1(h0, p0.reshape(NC, b, npg, HID),
              l0["W1"], row(l0["bn1_g"]), row(l0["bn1_b"]),
              l0["W2"], row(l0["obn_g"]), row(l0["obn_b"]))
    p1 = sweep(h1.reshape(n, HID), src2, dst2)
    rw1 = params["r_W1"]
    scores3, pooled3 = _tc2(
        h0, h1, p1.reshape(NC, b, npg, HID),
        l1["W1"], row(l1["bn1_g"]), row(l1["bn1_b"]),
        l1["W2"], row(l1["obn_g"]), row(l1["obn_b"]),
        rw1[:, :HID], rw1[:, HID:2 * HID], rw1[:, 2 * HID:],
        row(params["r_b1"]), jnp.pad(params["r_W2"], ((0, 7), (0, 0))),
        params["r_b2"].reshape(1, 1))
    return scores3.reshape(b, npg), pooled3.reshape(b, 1)
